# Initial kernel scaffold; baseline (speedup 1.0000x reference)
import jax
import jax.numpy as jnp
from jax import lax
from jax.experimental import pallas as pl
from jax.experimental.pallas import tpu as pltpu
from jax.experimental.pallas import tpu_sc as plsc

NUM_EMB = 1000000
DIM = 64
BATCH = 16384
HIST = 50

NC = 2
NS = 16
NW = NC * NS

B = BATCH * HIST
B_PER_W = B // NW
CHUNK = 128
NBUF = 8
N_CHUNKS = B_PER_W // CHUNK
assert N_CHUNKS % NBUF == 0


def _body(idx_hbm, table_hbm, out_hbm, idx_v, rows_v, g_sems, s_sems):
    wid = lax.axis_index("s") * NC + lax.axis_index("c")
    base = wid * B_PER_W

    pltpu.sync_copy(idx_hbm.at[pl.ds(base, B_PER_W)], idx_v)

    def start_gather(b, j):
        off = pl.multiple_of(j * CHUNK, CHUNK)
        pltpu.async_copy(
            table_hbm.at[idx_v.at[pl.ds(off, CHUNK)]], rows_v.at[b], g_sems[b]
        )

    def wait_gather(b):
        pltpu.make_async_copy(
            table_hbm.at[idx_v.at[pl.ds(0, CHUNK)]], rows_v.at[b], g_sems[b]
        ).wait()

    def start_store(b, j):
        row = pl.multiple_of(base + j * CHUNK, CHUNK)
        pltpu.async_copy(rows_v.at[b], out_hbm.at[pl.ds(row, CHUNK)], s_sems[b])

    def wait_store(b):
        pltpu.make_async_copy(
            rows_v.at[b], out_hbm.at[pl.ds(0, CHUNK)], s_sems[b]
        ).wait()

    for b in range(NBUF):
        start_gather(b, b)

    @pl.loop(0, N_CHUNKS - NBUF, step=NBUF)
    def _(g):
        for b in range(NBUF):
            j = g + b
            wait_gather(b)
            start_store(b, j)
            wait_store(b)
            start_gather(b, j + NBUF)

    for b in range(NBUF):
        j = N_CHUNKS - NBUF + b
        wait_gather(b)
        start_store(b, j)
        wait_store(b)


@jax.jit
def _embed(token_ids, W):
    idx = token_ids.reshape(B)
    mesh = plsc.VectorSubcoreMesh(core_axis_name="c", subcore_axis_name="s")
    out = pl.kernel(
        _body,
        out_type=jax.ShapeDtypeStruct((B, DIM), jnp.float32),
        mesh=mesh,
        scratch_types=[
            pltpu.VMEM((B_PER_W,), jnp.int32),
            pltpu.VMEM((NBUF, CHUNK, DIM), jnp.float32),
            [pltpu.SemaphoreType.DMA] * NBUF,
            [pltpu.SemaphoreType.DMA] * NBUF,
        ],
    )(idx, W)
    return out.reshape(BATCH, HIST, DIM)


def kernel(token_ids, W):
    return _embed(token_ids, W)


# speedup vs baseline: 1.8886x; 1.8886x over previous
import jax
import jax.numpy as jnp
from jax import lax
from jax.experimental import pallas as pl
from jax.experimental.pallas import tpu as pltpu
from jax.experimental.pallas import tpu_sc as plsc

NUM_EMB = 1000000
DIM = 64
BATCH = 16384
HIST = 50

NC = 2
NS = 16
NW = NC * NS

B = BATCH * HIST
B_PER_W = B // NW
CHUNK = 128
NBUF = 8
N_CHUNKS = B_PER_W // CHUNK
assert N_CHUNKS % NBUF == 0


def _body(idx_hbm, table_hbm, out_hbm, idx_v, rows_v, g_sems, s_sems):
    wid = lax.axis_index("s") * NC + lax.axis_index("c")
    base = wid * B_PER_W

    pltpu.sync_copy(idx_hbm.at[pl.ds(base, B_PER_W)], idx_v)

    def start_gather(b, j):
        off = pl.multiple_of(j * CHUNK, CHUNK)
        pltpu.async_copy(
            table_hbm.at[idx_v.at[pl.ds(off, CHUNK)]], rows_v.at[b], g_sems[b]
        )

    def wait_gather(b):
        pltpu.make_async_copy(
            table_hbm.at[idx_v.at[pl.ds(0, CHUNK)]], rows_v.at[b], g_sems[b]
        ).wait()

    def start_store(b, j):
        row = pl.multiple_of(base + j * CHUNK, CHUNK)
        pltpu.async_copy(rows_v.at[b], out_hbm.at[pl.ds(row, CHUNK)], s_sems[b])

    def wait_store(b):
        pltpu.make_async_copy(
            rows_v.at[b], out_hbm.at[pl.ds(0, CHUNK)], s_sems[b]
        ).wait()

    for b in range(NBUF):
        start_gather(b, b)

    @pl.loop(0, N_CHUNKS - NBUF, step=NBUF)
    def _(g):
        for b in range(NBUF):
            j = g + b
            wait_gather(b)
            start_store(b, j)
            wait_store(b)
            start_gather(b, j + NBUF)

    for b in range(NBUF):
        j = N_CHUNKS - NBUF + b
        wait_gather(b)
        start_store(b, j)
        wait_store(b)


@jax.jit
def _embed(token_ids, W):
    idx = token_ids.reshape(B)
    mesh = plsc.VectorSubcoreMesh(core_axis_name="c", subcore_axis_name="s")
    out = pl.kernel(
        _body,
        out_type=jax.ShapeDtypeStruct((B, DIM), jnp.float32),
        mesh=mesh,
        scratch_types=[
            pltpu.VMEM((B_PER_W,), jnp.int32),
            pltpu.VMEM((NBUF, CHUNK, DIM), jnp.float32),
            [pltpu.SemaphoreType.DMA] * NBUF,
            [pltpu.SemaphoreType.DMA] * NBUF,
        ],
        compiler_params=pltpu.CompilerParams(use_tc_tiling_on_sc=False),
    )(idx, W)
    return out.reshape(BATCH, HIST, DIM)


def kernel(token_ids, W):
    return _embed(token_ids, W)
